# baseline (speedup 1.0000x reference)
import functools

import jax
import jax.numpy as jnp
from jax import lax
from jax.experimental import pallas as pl
from jax.experimental.pallas import tpu as pltpu
from jax.experimental.pallas import tpu_sc as plsc

N = 10000
D = 128
DH = D // 2
NSC = 2
NSUB = 16
NW = NSC * NSUB
CH = 125
LCH = 128
PAD = 240
NACC = N + PAD
ZROWS = NACC // NSUB
BN = 1000

_mesh = plsc.VectorSubcoreMesh(
    core_axis_name="c", subcore_axis_name="s", num_cores=NSC)

_f32 = jnp.float32

_sc_params = pltpu.CompilerParams(use_tc_tiling_on_sc=False)


def _make_deg_kernel(nch):
  @functools.partial(
      pl.kernel,
      out_type=jax.ShapeDtypeStruct((NSC, NACC, 16), _f32),
      mesh=_mesh,
      scratch_types=[
          pltpu.VMEM((nch, CH), jnp.int32),
          pltpu.VMEM((CH, 16), _f32),
          pltpu.VMEM((128, 16), _f32),
          pltpu.VMEM_SHARED((NACC, 16), _f32),
      ],
      compiler_params=_sc_params,
  )
  def deg_kernel(dsts_hbm, out_hbm, dst_v, ones_v, zeros_v, deg_sh):
    c = lax.axis_index("c")
    s = lax.axis_index("s")
    wid = c * NSUB + s
    one = jnp.ones((16,), _f32)
    zero = jnp.zeros((16,), _f32)

    @pl.loop(0, CH)
    def _(r):
      ones_v[r, pl.ds(0, 16)] = one

    @pl.loop(0, 128)
    def _(r):
      zeros_v[r, pl.ds(0, 16)] = zero

    base = s * ZROWS
    for j in range(ZROWS // 128):
      pltpu.sync_copy(zeros_v, deg_sh.at[pl.ds(base + j * 128, 128)])
    pltpu.sync_copy(dsts_hbm.at[wid], dst_v)
    plsc.subcore_barrier()

    @pl.loop(0, nch)
    def _(ch):
      pltpu.sync_copy(ones_v, deg_sh.at[dst_v.at[ch]], add=True)

    plsc.subcore_barrier()
    pltpu.sync_copy(deg_sh.at[pl.ds(base, ZROWS)],
                    out_hbm.at[c, pl.ds(base, ZROWS)])

  return deg_kernel


def _make_scatter_kernel(nch, nlch):
  G = 4
  PH = 48
  phases = [min(PH, nch - p * PH) for p in range(-(-nch // PH))]

  @functools.partial(
      pl.kernel,
      out_type=jax.ShapeDtypeStruct((NACC, NSC, DH), _f32),
      mesh=_mesh,
      scratch_types=[
          pltpu.VMEM((PH, CH), jnp.int32),
          pltpu.VMEM((PH, CH), jnp.int32),
          pltpu.VMEM((nlch, LCH), jnp.int32),
          pltpu.VMEM((nlch, LCH), jnp.int32),
      ] + [pltpu.VMEM((LCH, DH), _f32) for _ in range(2 * G)] + [
          pltpu.VMEM_SHARED((NACC, DH), _f32),
          pltpu.SemaphoreType.DMA,
          pltpu.SemaphoreType.DMA,
          pltpu.SemaphoreType.DMA,
          pltpu.SemaphoreType.DMA,
      ],
      compiler_params=_sc_params,
  )
  def scatter_kernel(xs_hbm, srcs_hbm, dsts_hbm, lsrcs_hbm, ldsts_hbm,
                     out_hbm, src_v, dst_v, lsrc_v, ldst_v, *rest):
    bufs = rest[:2 * G]
    seta, setb = bufs[:G], bufs[G:]
    acc_sh, gsa, gsb, ssa, ssb = rest[2 * G:]
    c = lax.axis_index("c")
    s = lax.axis_index("s")
    zero = jnp.zeros((16,), _f32)
    buf0 = seta[0]

    @pl.loop(0, LCH)
    def _(r):
      for k in range(DH // 16):
        buf0[r, pl.ds(k * 16, 16)] = zero

    base = s * ZROWS
    for j in range(ZROWS // LCH):
      pltpu.sync_copy(buf0, acc_sh.at[pl.ds(base + j * LCH, LCH)])
    plsc.subcore_barrier()

    def fire_g(cb, st, sem):
      for j in range(G):
        pltpu.async_copy(xs_hbm.at[src_v.at[cb + j]],
                         st[j].at[pl.ds(0, CH)], sem)

    def wait_g(st, sem):
      for j in range(G):
        pltpu.make_async_copy(xs_hbm.at[src_v.at[0]],
                              st[j].at[pl.ds(0, CH)], sem).wait()

    def fire_s(cb, st, sem):
      for j in range(G):
        pltpu.async_copy(st[j].at[pl.ds(0, CH)],
                         acc_sh.at[dst_v.at[cb + j]], sem, add=True)

    def wait_s(st, sem):
      for j in range(G):
        pltpu.make_async_copy(st[j].at[pl.ds(0, CH)],
                              acc_sh.at[dst_v.at[0]], sem).wait()

    for p, plen in enumerate(phases):
      pltpu.sync_copy(srcs_hbm.at[c, s, pl.ds(p * PH, plen)],
                      src_v.at[pl.ds(0, plen)])
      pltpu.sync_copy(dsts_hbm.at[s, pl.ds(p * PH, plen)],
                      dst_v.at[pl.ds(0, plen)])

      ngrp = (plen // (2 * G)) * 2
      if ngrp >= 4:
        fire_g(0, seta, gsa)
        fire_g(G, setb, gsb)
        wait_g(seta, gsa)
        fire_s(0, seta, ssa)
        wait_g(setb, gsb)
        fire_s(G, setb, ssb)
        wait_s(seta, ssa)
        fire_g(2 * G, seta, gsa)

        @pl.loop(2 * G, ngrp * G, step=2 * G)
        def _(cb):
          wait_g(seta, gsa)
          fire_s(cb, seta, ssa)
          wait_s(setb, ssb)
          fire_g(cb + G, setb, gsb)
          wait_g(setb, gsb)
          fire_s(cb + G, setb, ssb)
          wait_s(seta, ssa)

          @pl.when(cb + 2 * G < ngrp * G)
          def _():
            fire_g(cb + 2 * G, seta, gsa)

        wait_s(setb, ssb)
        done = ngrp * G
      else:
        done = 0

      for ch0 in range(done, plen):
        st = seta[ch0 % G]
        pltpu.async_copy(xs_hbm.at[src_v.at[ch0]],
                         st.at[pl.ds(0, CH)], gsa).wait()
        pltpu.sync_copy(st.at[pl.ds(0, CH)],
                        acc_sh.at[dst_v.at[ch0]], add=True)

    pltpu.sync_copy(lsrcs_hbm.at[c, s], lsrc_v)
    pltpu.sync_copy(ldsts_hbm.at[s], ldst_v)
    for ch0 in range(nlch):
      st = seta[ch0 % G]
      pltpu.async_copy(xs_hbm.at[lsrc_v.at[ch0]], st, gsa).wait()
      pltpu.sync_copy(st, acc_sh.at[ldst_v.at[ch0]], add=True)

    plsc.subcore_barrier()
    pltpu.sync_copy(acc_sh.at[pl.ds(base, ZROWS)],
                    out_hbm.at[pl.ds(base, ZROWS), c])

  return scatter_kernel


def _dot(a, b):
  return lax.dot_general(a, b, (((1,), (0,)), ((), ())),
                         precision=lax.Precision.HIGHEST,
                         preferred_element_type=_f32)


def _mm_scale_body(x_ref, w_ref, degp_ref, xs_ref, dinv_ref):
  d = degp_ref[0, :, 0:1] + degp_ref[1, :, 0:1] + 1.0
  dinv = lax.rsqrt(d)
  xs_ref[...] = _dot(x_ref[...], w_ref[...]) * dinv
  dinv_ref[...] = dinv


def _tc_mm_scale(x, w, degp):
  return pl.pallas_call(
      _mm_scale_body,
      grid=(N // BN,),
      in_specs=[pl.BlockSpec((BN, D), lambda i: (i, 0)),
                pl.BlockSpec((D, D), lambda i: (0, 0)),
                pl.BlockSpec((NSC, BN, 16), lambda i: (0, i, 0))],
      out_specs=[pl.BlockSpec((BN, D), lambda i: (i, 0)),
                 pl.BlockSpec((BN, 1), lambda i: (i, 0))],
      out_shape=[jax.ShapeDtypeStruct((N, D), _f32),
                 jax.ShapeDtypeStruct((N, 1), _f32)],
  )(x, w, degp)


def _mid_body(y_ref, dinv_ref, b_ref, w_ref, o_ref):
  dinv = dinv_ref[...]
  h = y_ref[...] * dinv + b_ref[...]
  o_ref[...] = _dot(h, w_ref[...]) * dinv


def _tc_mid(y, dinv, b, w):
  return pl.pallas_call(
      _mid_body,
      grid=(N // BN,),
      in_specs=[pl.BlockSpec((BN, D), lambda i: (i, 0)),
                pl.BlockSpec((BN, 1), lambda i: (i, 0)),
                pl.BlockSpec((1, D), lambda i: (0, 0)),
                pl.BlockSpec((D, D), lambda i: (0, 0))],
      out_specs=pl.BlockSpec((BN, D), lambda i: (i, 0)),
      out_shape=jax.ShapeDtypeStruct((N, D), _f32),
  )(y, dinv, b, w)


def _fin_body(y_ref, dinv_ref, b_ref, o_ref):
  o_ref[...] = y_ref[...] * dinv_ref[...] + b_ref[...]


def _tc_fin(y, dinv, b):
  return pl.pallas_call(
      _fin_body,
      grid=(N // BN,),
      in_specs=[pl.BlockSpec((BN, D), lambda i: (i, 0)),
                pl.BlockSpec((BN, 1), lambda i: (i, 0)),
                pl.BlockSpec((1, D), lambda i: (0, 0))],
      out_specs=pl.BlockSpec((BN, D), lambda i: (i, 0)),
      out_shape=jax.ShapeDtypeStruct((N, D), _f32),
  )(y, dinv, b)


def kernel(in_feat, adj_t, W1, b1, W2, b2):
  e = adj_t.shape[1]
  if e % (NSUB * CH) or e % (NW * CH) or N % (NSUB * CH):
    raise ValueError("unsupported edge/node count for this kernel")
  nch = e // (NSUB * CH)
  nch32 = e // (NW * CH)
  nlch = -(-(N // NSUB) // LCH)

  src = adj_t[0]
  dst = adj_t[1]
  srcs2 = jnp.stack([2 * src, 2 * src + 1]).reshape(NSC, NSUB, nch, CH)
  dsts = dst.reshape(NSUB, nch, CH)
  degdsts = dst.reshape(NW, nch32, CH)

  r = jnp.arange(NSUB * nlch * LCH, dtype=jnp.int32)
  npr = N // NSUB
  sub, off = r // (nlch * LCH), r % (nlch * LCH)
  node = sub * npr + off
  valid = off < npr
  lsrc0 = jnp.where(valid, 2 * node, 0)
  ldst = jnp.where(valid, node, N + (r % PAD))
  lsrcs = jnp.stack([lsrc0, lsrc0 + 1]).reshape(NSC, NSUB, nlch, LCH)
  ldsts = ldst.reshape(NSUB, nlch, LCH)

  deg_k = _make_deg_kernel(nch32)
  scat_k = _make_scatter_kernel(nch, nlch)

  degp = deg_k(degdsts)
  xs1, dinv = _tc_mm_scale(in_feat, W1, degp)
  y1 = scat_k(xs1.reshape(NSC * N, DH), srcs2, dsts, lsrcs, ldsts)
  xs2 = _tc_mid(y1.reshape(NACC, D), dinv, b1.reshape(1, D), W2)
  y2 = scat_k(xs2.reshape(NSC * N, DH), srcs2, dsts, lsrcs, ldsts)
  return _tc_fin(y2.reshape(NACC, D), dinv, b2.reshape(1, D))


# speedup vs baseline: 1.2132x; 1.2132x over previous
import functools

import jax
import jax.numpy as jnp
from jax import lax
from jax.experimental import pallas as pl
from jax.experimental.pallas import tpu as pltpu
from jax.experimental.pallas import tpu_sc as plsc

N = 10000
D = 128
DH = D // 2
NSC = 2
NSUB = 16
NW = NSC * NSUB
CH = 125
LCH = 128
PAD = 240
NACC = N + PAD
ZROWS = NACC // NSUB
BN = 1000

_mesh = plsc.VectorSubcoreMesh(
    core_axis_name="c", subcore_axis_name="s", num_cores=NSC)

_f32 = jnp.float32

_sc_params = pltpu.CompilerParams(use_tc_tiling_on_sc=False)


def _make_deg_kernel(nch):
  @functools.partial(
      pl.kernel,
      out_type=jax.ShapeDtypeStruct((NSC, NACC, 16), _f32),
      mesh=_mesh,
      scratch_types=[
          pltpu.VMEM((nch, CH), jnp.int32),
          pltpu.VMEM((CH, 16), _f32),
          pltpu.VMEM((128, 16), _f32),
          pltpu.VMEM_SHARED((NACC, 16), _f32),
      ],
      compiler_params=_sc_params,
  )
  def deg_kernel(dsts_hbm, out_hbm, dst_v, ones_v, zeros_v, deg_sh):
    c = lax.axis_index("c")
    s = lax.axis_index("s")
    wid = c * NSUB + s
    one = jnp.ones((16,), _f32)
    zero = jnp.zeros((16,), _f32)

    @pl.loop(0, CH)
    def _(r):
      ones_v[r, pl.ds(0, 16)] = one

    @pl.loop(0, 128)
    def _(r):
      zeros_v[r, pl.ds(0, 16)] = zero

    base = s * ZROWS
    for j in range(ZROWS // 128):
      pltpu.sync_copy(zeros_v, deg_sh.at[pl.ds(base + j * 128, 128)])
    pltpu.sync_copy(dsts_hbm.at[wid], dst_v)
    plsc.subcore_barrier()

    @pl.loop(0, nch)
    def _(ch):
      pltpu.sync_copy(ones_v, deg_sh.at[dst_v.at[ch]], add=True)

    plsc.subcore_barrier()
    pltpu.sync_copy(deg_sh.at[pl.ds(base, ZROWS)],
                    out_hbm.at[c, pl.ds(base, ZROWS)])

  return deg_kernel


def _make_scatter_kernel(nch, nlch):
  G = 4
  PH = 48
  phases = [min(PH, nch - p * PH) for p in range(-(-nch // PH))]

  @functools.partial(
      pl.kernel,
      out_type=jax.ShapeDtypeStruct((NACC, D), _f32),
      mesh=_mesh,
      scratch_types=[
          pltpu.VMEM((PH, CH), jnp.int32),
          pltpu.VMEM((PH, CH), jnp.int32),
          pltpu.VMEM((nlch, LCH), jnp.int32),
          pltpu.VMEM((nlch, LCH), jnp.int32),
      ] + [pltpu.VMEM((LCH, DH), _f32) for _ in range(2 * G)] + [
          pltpu.VMEM_SHARED((NACC, DH), _f32),
          pltpu.SemaphoreType.DMA,
          pltpu.SemaphoreType.DMA,
          pltpu.SemaphoreType.DMA,
          pltpu.SemaphoreType.DMA,
      ],
      compiler_params=_sc_params,
  )
  def scatter_kernel(xs_hbm, srcs_hbm, dsts_hbm, lsrcs_hbm, ldsts_hbm,
                     out_hbm, src_v, dst_v, lsrc_v, ldst_v, *rest):
    bufs = rest[:2 * G]
    seta, setb = bufs[:G], bufs[G:]
    acc_sh, gsa, gsb, ssa, ssb = rest[2 * G:]
    c = lax.axis_index("c")
    s = lax.axis_index("s")
    zero = jnp.zeros((16,), _f32)
    buf0 = seta[0]

    @pl.loop(0, LCH)
    def _(r):
      for k in range(DH // 16):
        buf0[r, pl.ds(k * 16, 16)] = zero

    base = s * ZROWS
    for j in range(ZROWS // LCH):
      pltpu.sync_copy(buf0, acc_sh.at[pl.ds(base + j * LCH, LCH)])
    plsc.subcore_barrier()

    def fire_g(cb, st, sem):
      for j in range(G):
        pltpu.async_copy(xs_hbm.at[src_v.at[cb + j]],
                         st[j].at[pl.ds(0, CH)], sem)

    def wait_g(st, sem):
      for j in range(G):
        pltpu.make_async_copy(xs_hbm.at[src_v.at[0]],
                              st[j].at[pl.ds(0, CH)], sem).wait()

    def fire_s(cb, st, sem):
      for j in range(G):
        pltpu.async_copy(st[j].at[pl.ds(0, CH)],
                         acc_sh.at[dst_v.at[cb + j]], sem, add=True)

    def wait_s(st, sem):
      for j in range(G):
        pltpu.make_async_copy(st[j].at[pl.ds(0, CH)],
                              acc_sh.at[dst_v.at[0]], sem).wait()

    for p, plen in enumerate(phases):
      pltpu.sync_copy(srcs_hbm.at[c, s, pl.ds(p * PH, plen)],
                      src_v.at[pl.ds(0, plen)])
      pltpu.sync_copy(dsts_hbm.at[s, pl.ds(p * PH, plen)],
                      dst_v.at[pl.ds(0, plen)])

      ngrp = (plen // (2 * G)) * 2
      if ngrp >= 4:
        fire_g(0, seta, gsa)
        fire_g(G, setb, gsb)
        wait_g(seta, gsa)
        fire_s(0, seta, ssa)
        wait_g(setb, gsb)
        fire_s(G, setb, ssb)
        wait_s(seta, ssa)
        fire_g(2 * G, seta, gsa)

        @pl.loop(2 * G, ngrp * G, step=2 * G)
        def _(cb):
          wait_g(seta, gsa)
          fire_s(cb, seta, ssa)
          wait_s(setb, ssb)
          fire_g(cb + G, setb, gsb)
          wait_g(setb, gsb)
          fire_s(cb + G, setb, ssb)
          wait_s(seta, ssa)

          @pl.when(cb + 2 * G < ngrp * G)
          def _():
            fire_g(cb + 2 * G, seta, gsa)

        wait_s(setb, ssb)
        done = ngrp * G
      else:
        done = 0

      for ch0 in range(done, plen):
        st = seta[ch0 % G]
        pltpu.async_copy(xs_hbm.at[src_v.at[ch0]],
                         st.at[pl.ds(0, CH)], gsa).wait()
        pltpu.sync_copy(st.at[pl.ds(0, CH)],
                        acc_sh.at[dst_v.at[ch0]], add=True)

    pltpu.sync_copy(lsrcs_hbm.at[c, s], lsrc_v)
    pltpu.sync_copy(ldsts_hbm.at[s], ldst_v)
    for ch0 in range(nlch):
      st = seta[ch0 % G]
      pltpu.async_copy(xs_hbm.at[lsrc_v.at[ch0]], st, gsa).wait()
      pltpu.sync_copy(st, acc_sh.at[ldst_v.at[ch0]], add=True)

    plsc.subcore_barrier()
    pltpu.sync_copy(acc_sh.at[pl.ds(base, ZROWS)],
                    out_hbm.at[pl.ds(base, ZROWS), pl.ds(c * DH, DH)])

  return scatter_kernel


def _dot(a, b):
  return lax.dot_general(a, b, (((1,), (0,)), ((), ())),
                         precision=lax.Precision.HIGHEST,
                         preferred_element_type=_f32)


def _mm_scale_body(x_ref, w_ref, degp_ref, xs_ref, dinv_ref):
  d = degp_ref[0, :, 0:1] + degp_ref[1, :, 0:1] + 1.0
  dinv = lax.rsqrt(d)
  xs_ref[...] = _dot(x_ref[...], w_ref[...]) * dinv
  dinv_ref[...] = dinv


def _tc_mm_scale(x, w, degp):
  return pl.pallas_call(
      _mm_scale_body,
      grid=(N // BN,),
      in_specs=[pl.BlockSpec((BN, D), lambda i: (i, 0)),
                pl.BlockSpec((D, D), lambda i: (0, 0)),
                pl.BlockSpec((NSC, BN, 16), lambda i: (0, i, 0))],
      out_specs=[pl.BlockSpec((BN, D), lambda i: (i, 0)),
                 pl.BlockSpec((BN, 1), lambda i: (i, 0))],
      out_shape=[jax.ShapeDtypeStruct((N, D), _f32),
                 jax.ShapeDtypeStruct((N, 1), _f32)],
  )(x, w, degp)


def _mid_body(y_ref, dinv_ref, b_ref, w_ref, o_ref):
  dinv = dinv_ref[...]
  h = y_ref[...] * dinv + b_ref[...]
  o_ref[...] = _dot(h, w_ref[...]) * dinv


def _tc_mid(y, dinv, b, w):
  return pl.pallas_call(
      _mid_body,
      grid=(N // BN,),
      in_specs=[pl.BlockSpec((BN, D), lambda i: (i, 0)),
                pl.BlockSpec((BN, 1), lambda i: (i, 0)),
                pl.BlockSpec((1, D), lambda i: (0, 0)),
                pl.BlockSpec((D, D), lambda i: (0, 0))],
      out_specs=pl.BlockSpec((BN, D), lambda i: (i, 0)),
      out_shape=jax.ShapeDtypeStruct((N, D), _f32),
  )(y, dinv, b, w)


def _fin_body(y_ref, dinv_ref, b_ref, o_ref):
  o_ref[...] = y_ref[...] * dinv_ref[...] + b_ref[...]


def _tc_fin(y, dinv, b):
  return pl.pallas_call(
      _fin_body,
      grid=(N // BN,),
      in_specs=[pl.BlockSpec((BN, D), lambda i: (i, 0)),
                pl.BlockSpec((BN, 1), lambda i: (i, 0)),
                pl.BlockSpec((1, D), lambda i: (0, 0))],
      out_specs=pl.BlockSpec((BN, D), lambda i: (i, 0)),
      out_shape=jax.ShapeDtypeStruct((N, D), _f32),
  )(y, dinv, b)


def kernel(in_feat, adj_t, W1, b1, W2, b2):
  e = adj_t.shape[1]
  if e % (NSUB * CH) or e % (NW * CH) or N % (NSUB * CH):
    raise ValueError("unsupported edge/node count for this kernel")
  nch = e // (NSUB * CH)
  nch32 = e // (NW * CH)
  nlch = -(-(N // NSUB) // LCH)

  src = adj_t[0]
  dst = adj_t[1]
  srcs2 = jnp.stack([2 * src, 2 * src + 1]).reshape(NSC, NSUB, nch, CH)
  dsts = dst.reshape(NSUB, nch, CH)
  degdsts = dst.reshape(NW, nch32, CH)

  r = jnp.arange(NSUB * nlch * LCH, dtype=jnp.int32)
  npr = N // NSUB
  sub, off = r // (nlch * LCH), r % (nlch * LCH)
  node = sub * npr + off
  valid = off < npr
  lsrc0 = jnp.where(valid, 2 * node, 0)
  lsrcs = jnp.stack([lsrc0, lsrc0 + 1]).reshape(NSC, NSUB, nlch, LCH)
  ldsts = jnp.where(valid, node, N + (r % PAD)).reshape(NSUB, nlch, LCH)

  deg_k = _make_deg_kernel(nch32)
  scat_k = _make_scatter_kernel(nch, nlch)

  degp = deg_k(degdsts)
  xs1, dinv = _tc_mm_scale(in_feat, W1, degp)
  y1 = scat_k(xs1.reshape(NSC * N, DH), srcs2, dsts, lsrcs, ldsts)
  xs2 = _tc_mid(y1, dinv, b1.reshape(1, D), W2)
  y2 = scat_k(xs2.reshape(NSC * N, DH), srcs2, dsts, lsrcs, ldsts)
  return _tc_fin(y2, dinv, b2.reshape(1, D))


# speedup vs baseline: 1.2695x; 1.0465x over previous
import functools

import jax
import jax.numpy as jnp
from jax import lax
from jax.experimental import pallas as pl
from jax.experimental.pallas import tpu as pltpu
from jax.experimental.pallas import tpu_sc as plsc

N = 10000
D = 128
DH = D // 2
NSC = 2
NSUB = 16
NW = NSC * NSUB
CH = 80
NACC = 10240
ZROWS = NACC // NSUB
BN = 1000

_mesh = plsc.VectorSubcoreMesh(
    core_axis_name="c", subcore_axis_name="s", num_cores=NSC)

_f32 = jnp.float32

_sc_params = pltpu.CompilerParams(use_tc_tiling_on_sc=False)


def _make_deg_kernel(nch):
  @functools.partial(
      pl.kernel,
      out_type=jax.ShapeDtypeStruct((NSC, NACC, 16), _f32),
      mesh=_mesh,
      scratch_types=[
          pltpu.VMEM((nch, CH), jnp.int32),
          pltpu.VMEM((CH, 16), _f32),
          pltpu.VMEM((128, 16), _f32),
          pltpu.VMEM_SHARED((NACC, 16), _f32),
      ],
      compiler_params=_sc_params,
  )
  def deg_kernel(dsts_hbm, out_hbm, dst_v, ones_v, zeros_v, deg_sh):
    c = lax.axis_index("c")
    s = lax.axis_index("s")
    wid = c * NSUB + s
    one = jnp.ones((16,), _f32)
    zero = jnp.zeros((16,), _f32)

    @pl.loop(0, CH)
    def _(r):
      ones_v[r, pl.ds(0, 16)] = one

    @pl.loop(0, 128)
    def _(r):
      zeros_v[r, pl.ds(0, 16)] = zero

    base = s * ZROWS
    for j in range(ZROWS // 128):
      pltpu.sync_copy(zeros_v, deg_sh.at[pl.ds(base + j * 128, 128)])
    pltpu.sync_copy(dsts_hbm.at[wid], dst_v)
    plsc.subcore_barrier()

    @pl.loop(0, nch)
    def _(ch):
      pltpu.sync_copy(ones_v, deg_sh.at[dst_v.at[ch]], add=True)

    plsc.subcore_barrier()
    pltpu.sync_copy(deg_sh.at[pl.ds(base, ZROWS)],
                    out_hbm.at[c, pl.ds(base, ZROWS)])

  return deg_kernel


def _make_scatter_kernel(nch):
  G = 4
  PH = 64
  phases = [min(PH, nch - p * PH) for p in range(-(-nch // PH))]

  @functools.partial(
      pl.kernel,
      out_type=jax.ShapeDtypeStruct((NACC, D), _f32),
      mesh=_mesh,
      scratch_types=[
          pltpu.VMEM((PH, CH), jnp.int32),
          pltpu.VMEM((PH, CH), jnp.int32),
      ] + [pltpu.VMEM((CH, DH), _f32) for _ in range(2 * G)] + [
          pltpu.VMEM_SHARED((NACC, DH), _f32),
          pltpu.SemaphoreType.DMA,
          pltpu.SemaphoreType.DMA,
          pltpu.SemaphoreType.DMA,
          pltpu.SemaphoreType.DMA,
      ],
      compiler_params=_sc_params,
  )
  def scatter_kernel(xs_hbm, srcs_hbm, dsts_hbm,
                     out_hbm, src_v, dst_v, *rest):
    bufs = rest[:2 * G]
    seta, setb = bufs[:G], bufs[G:]
    acc_sh, gsa, gsb, ssa, ssb = rest[2 * G:]
    c = lax.axis_index("c")
    s = lax.axis_index("s")
    cvec = lax.broadcast(c, (16,))
    zero = jnp.zeros((16,), _f32)
    buf0 = seta[0]

    @pl.loop(0, CH)
    def _(r):
      for k in range(DH // 16):
        buf0[r, pl.ds(k * 16, 16)] = zero

    base = s * ZROWS
    for j in range(ZROWS // CH):
      pltpu.sync_copy(buf0, acc_sh.at[pl.ds(base + j * CH, CH)])
    plsc.subcore_barrier()

    def fire_g(cb, st, sem):
      for j in range(G):
        pltpu.async_copy(xs_hbm.at[src_v.at[cb + j]], st[j], sem)

    def wait_g(st, sem):
      for j in range(G):
        pltpu.make_async_copy(xs_hbm.at[src_v.at[0]], st[j], sem).wait()

    def fire_s(cb, st, sem):
      for j in range(G):
        pltpu.async_copy(st[j], acc_sh.at[dst_v.at[cb + j]], sem, add=True)

    def wait_s(st, sem):
      for j in range(G):
        pltpu.make_async_copy(st[j], acc_sh.at[dst_v.at[0]], sem).wait()

    for p, plen in enumerate(phases):
      pltpu.sync_copy(srcs_hbm.at[s, pl.ds(p * PH, plen)],
                      src_v.at[pl.ds(0, plen)])
      pltpu.sync_copy(dsts_hbm.at[s, pl.ds(p * PH, plen)],
                      dst_v.at[pl.ds(0, plen)])

      @pl.loop(0, plen)
      def _(rr):
        for k in range(CH // 16):
          v = src_v[rr, pl.ds(k * 16, 16)]
          src_v[rr, pl.ds(k * 16, 16)] = v + v + cvec

      ngrp = (plen // (2 * G)) * 2
      if ngrp >= 4:
        fire_g(0, seta, gsa)
        fire_g(G, setb, gsb)
        wait_g(seta, gsa)
        fire_s(0, seta, ssa)
        wait_g(setb, gsb)
        fire_s(G, setb, ssb)
        wait_s(seta, ssa)
        fire_g(2 * G, seta, gsa)

        @pl.loop(2 * G, ngrp * G, step=2 * G)
        def _(cb):
          wait_g(seta, gsa)
          fire_s(cb, seta, ssa)
          wait_s(setb, ssb)
          fire_g(cb + G, setb, gsb)
          wait_g(setb, gsb)
          fire_s(cb + G, setb, ssb)
          wait_s(seta, ssa)

          @pl.when(cb + 2 * G < ngrp * G)
          def _():
            fire_g(cb + 2 * G, seta, gsa)

        wait_s(setb, ssb)
        done = ngrp * G
      else:
        done = 0

      for ch0 in range(done, plen):
        st = seta[ch0 % G]
        pltpu.async_copy(xs_hbm.at[src_v.at[ch0]], st, gsa).wait()
        pltpu.sync_copy(st, acc_sh.at[dst_v.at[ch0]], add=True)

    plsc.subcore_barrier()
    pltpu.sync_copy(acc_sh.at[pl.ds(base, ZROWS)],
                    out_hbm.at[pl.ds(base, ZROWS), pl.ds(c * DH, DH)])

  return scatter_kernel


def _dot(a, b):
  return lax.dot_general(a, b, (((1,), (0,)), ((), ())),
                         precision=lax.Precision.HIGHEST,
                         preferred_element_type=_f32)


def _mm_scale_body(x_ref, w_ref, degp_ref, xs_ref, dinv_ref):
  d = degp_ref[0, :, 0:1] + degp_ref[1, :, 0:1] + 1.0
  dinv = lax.rsqrt(d)
  xs_ref[...] = _dot(x_ref[...], w_ref[...]) * dinv
  dinv_ref[...] = dinv


def _tc_mm_scale(x, w, degp):
  return pl.pallas_call(
      _mm_scale_body,
      grid=(N // BN,),
      in_specs=[pl.BlockSpec((BN, D), lambda i: (i, 0)),
                pl.BlockSpec((D, D), lambda i: (0, 0)),
                pl.BlockSpec((NSC, BN, 16), lambda i: (0, i, 0))],
      out_specs=[pl.BlockSpec((BN, D), lambda i: (i, 0)),
                 pl.BlockSpec((BN, 1), lambda i: (i, 0))],
      out_shape=[jax.ShapeDtypeStruct((N, D), _f32),
                 jax.ShapeDtypeStruct((N, 1), _f32)],
  )(x, w, degp)


def _mid_body(y_ref, xs_ref, dinv_ref, b_ref, w_ref, o_ref):
  dinv = dinv_ref[...]
  h = (y_ref[...] + xs_ref[...]) * dinv + b_ref[...]
  o_ref[...] = _dot(h, w_ref[...]) * dinv


def _tc_mid(y, xs, dinv, b, w):
  return pl.pallas_call(
      _mid_body,
      grid=(N // BN,),
      in_specs=[pl.BlockSpec((BN, D), lambda i: (i, 0)),
                pl.BlockSpec((BN, D), lambda i: (i, 0)),
                pl.BlockSpec((BN, 1), lambda i: (i, 0)),
                pl.BlockSpec((1, D), lambda i: (0, 0)),
                pl.BlockSpec((D, D), lambda i: (0, 0))],
      out_specs=pl.BlockSpec((BN, D), lambda i: (i, 0)),
      out_shape=jax.ShapeDtypeStruct((N, D), _f32),
  )(y, xs, dinv, b, w)


def _fin_body(y_ref, xs_ref, dinv_ref, b_ref, o_ref):
  o_ref[...] = (y_ref[...] + xs_ref[...]) * dinv_ref[...] + b_ref[...]


def _tc_fin(y, xs, dinv, b):
  return pl.pallas_call(
      _fin_body,
      grid=(N // BN,),
      in_specs=[pl.BlockSpec((BN, D), lambda i: (i, 0)),
                pl.BlockSpec((BN, D), lambda i: (i, 0)),
                pl.BlockSpec((BN, 1), lambda i: (i, 0)),
                pl.BlockSpec((1, D), lambda i: (0, 0))],
      out_specs=pl.BlockSpec((BN, D), lambda i: (i, 0)),
      out_shape=jax.ShapeDtypeStruct((N, D), _f32),
  )(y, xs, dinv, b)


def kernel(in_feat, adj_t, W1, b1, W2, b2):
  e = adj_t.shape[1]
  if e % (NSUB * CH) or e % (NW * CH):
    raise ValueError("unsupported edge count for this kernel")
  nch = e // (NSUB * CH)
  nch32 = e // (NW * CH)

  srcs = adj_t[0].reshape(NSUB, nch, CH)
  dsts = adj_t[1].reshape(NSUB, nch, CH)
  degdsts = adj_t[1].reshape(NW, nch32, CH)

  deg_k = _make_deg_kernel(nch32)
  scat_k = _make_scatter_kernel(nch)

  degp = deg_k(degdsts)
  xs1, dinv = _tc_mm_scale(in_feat, W1, degp)
  y1 = scat_k(xs1.reshape(NSC * N, DH), srcs, dsts)
  xs2 = _tc_mid(y1, xs1, dinv, b1.reshape(1, D), W2)
  y2 = scat_k(xs2.reshape(NSC * N, DH), srcs, dsts)
  return _tc_fin(y2, xs2, dinv, b2.reshape(1, D))


# speedup vs baseline: 1.3815x; 1.0882x over previous
import functools

import jax
import jax.numpy as jnp
from jax import lax
from jax.experimental import pallas as pl
from jax.experimental.pallas import tpu as pltpu
from jax.experimental.pallas import tpu_sc as plsc

N = 10000
D = 128
DH = D // 2
NSC = 2
NSUB = 16
NW = NSC * NSUB
CH = 80
NACC = 10240
ZROWS = NACC // NSUB
BN = 2000

_mesh = plsc.VectorSubcoreMesh(
    core_axis_name="c", subcore_axis_name="s", num_cores=NSC)

_f32 = jnp.float32

_sc_params = pltpu.CompilerParams(use_tc_tiling_on_sc=False)


def _make_deg_kernel(nch):
  @functools.partial(
      pl.kernel,
      out_type=jax.ShapeDtypeStruct((NSC, NACC, 16), _f32),
      mesh=_mesh,
      scratch_types=[
          pltpu.VMEM((nch, CH), jnp.int32),
          pltpu.VMEM((CH, 16), _f32),
          pltpu.VMEM((128, 16), _f32),
          pltpu.SemaphoreType.DMA,
          pltpu.VMEM_SHARED((NACC, 16), _f32),
      ],
      compiler_params=_sc_params,
  )
  def deg_kernel(adj_hbm, out_hbm, dst_v, ones_v, zeros_v, sem, deg_sh):
    c = lax.axis_index("c")
    s = lax.axis_index("s")
    one = jnp.ones((16,), _f32)
    zero = jnp.zeros((16,), _f32)

    @pl.loop(0, CH)
    def _(r):
      ones_v[r, pl.ds(0, 16)] = one

    @pl.loop(0, 128)
    def _(r):
      zeros_v[r, pl.ds(0, 16)] = zero

    base = s * ZROWS
    for j in range(ZROWS // 128):
      pltpu.sync_copy(zeros_v, deg_sh.at[pl.ds(base + j * 128, 128)])
    pltpu.sync_copy(adj_hbm.at[1, s, pl.ds(c * nch, nch)], dst_v)
    plsc.subcore_barrier()

    NB = 8
    nfull = (nch // NB) * NB

    @pl.loop(0, nfull, step=NB)
    def _(cb):
      for j in range(NB):
        pltpu.async_copy(ones_v, deg_sh.at[dst_v.at[cb + j]], sem, add=True)
      for j in range(NB):
        pltpu.make_async_copy(ones_v, deg_sh.at[dst_v.at[0]], sem).wait()

    for ch in range(nfull, nch):
      pltpu.sync_copy(ones_v, deg_sh.at[dst_v.at[ch]], add=True)

    plsc.subcore_barrier()
    pltpu.sync_copy(deg_sh.at[pl.ds(base, ZROWS)],
                    out_hbm.at[c, pl.ds(base, ZROWS)])

  return deg_kernel


def _make_scatter_kernel(nch):
  G = 4
  PH = 64
  phases = [min(PH, nch - p * PH) for p in range(-(-nch // PH))]

  @functools.partial(
      pl.kernel,
      out_type=jax.ShapeDtypeStruct((NACC, D), _f32),
      mesh=_mesh,
      scratch_types=[
          pltpu.VMEM((PH, CH), jnp.int32),
          pltpu.VMEM((PH, CH), jnp.int32),
      ] + [pltpu.VMEM((CH, DH), _f32) for _ in range(2 * G)] + [
          pltpu.VMEM_SHARED((NACC, DH), _f32),
          pltpu.SemaphoreType.DMA,
          pltpu.SemaphoreType.DMA,
          pltpu.SemaphoreType.DMA,
          pltpu.SemaphoreType.DMA,
      ],
      compiler_params=_sc_params,
  )
  def scatter_kernel(xs_hbm, adj_hbm, out_hbm, src_v, dst_v, *rest):
    bufs = rest[:2 * G]
    seta, setb = bufs[:G], bufs[G:]
    acc_sh, gsa, gsb, ssa, ssb = rest[2 * G:]
    c = lax.axis_index("c")
    s = lax.axis_index("s")
    cvec = lax.broadcast(c, (16,))
    zero = jnp.zeros((16,), _f32)
    buf0 = seta[0]

    @pl.loop(0, CH)
    def _(r):
      for k in range(DH // 16):
        buf0[r, pl.ds(k * 16, 16)] = zero

    base = s * ZROWS
    for j in range(ZROWS // CH):
      pltpu.sync_copy(buf0, acc_sh.at[pl.ds(base + j * CH, CH)])
    plsc.subcore_barrier()

    def fire_g(cb, st, sem):
      for j in range(G):
        pltpu.async_copy(xs_hbm.at[src_v.at[cb + j]], st[j], sem)

    def wait_g(st, sem):
      for j in range(G):
        pltpu.make_async_copy(xs_hbm.at[src_v.at[0]], st[j], sem).wait()

    def fire_s(cb, st, sem):
      for j in range(G):
        pltpu.async_copy(st[j], acc_sh.at[dst_v.at[cb + j]], sem, add=True)

    def wait_s(st, sem):
      for j in range(G):
        pltpu.make_async_copy(st[j], acc_sh.at[dst_v.at[0]], sem).wait()

    for p, plen in enumerate(phases):
      pltpu.sync_copy(adj_hbm.at[0, s, pl.ds(p * PH, plen)],
                      src_v.at[pl.ds(0, plen)])
      pltpu.sync_copy(adj_hbm.at[1, s, pl.ds(p * PH, plen)],
                      dst_v.at[pl.ds(0, plen)])

      @pl.loop(0, plen)
      def _(rr):
        for k in range(CH // 16):
          v = src_v[rr, pl.ds(k * 16, 16)]
          src_v[rr, pl.ds(k * 16, 16)] = v + v + cvec

      ngrp = (plen // (2 * G)) * 2
      if ngrp >= 4:
        fire_g(0, seta, gsa)
        fire_g(G, setb, gsb)
        wait_g(seta, gsa)
        fire_s(0, seta, ssa)
        wait_g(setb, gsb)
        fire_s(G, setb, ssb)
        wait_s(seta, ssa)
        fire_g(2 * G, seta, gsa)

        @pl.loop(2 * G, ngrp * G, step=2 * G)
        def _(cb):
          wait_g(seta, gsa)
          fire_s(cb, seta, ssa)
          wait_s(setb, ssb)
          fire_g(cb + G, setb, gsb)
          wait_g(setb, gsb)
          fire_s(cb + G, setb, ssb)
          wait_s(seta, ssa)

          @pl.when(cb + 2 * G < ngrp * G)
          def _():
            fire_g(cb + 2 * G, seta, gsa)

        wait_s(setb, ssb)
        done = ngrp * G
      else:
        done = 0

      for ch0 in range(done, plen):
        st = seta[ch0 % G]
        pltpu.async_copy(xs_hbm.at[src_v.at[ch0]], st, gsa).wait()
        pltpu.sync_copy(st, acc_sh.at[dst_v.at[ch0]], add=True)

    plsc.subcore_barrier()
    pltpu.sync_copy(acc_sh.at[pl.ds(base, ZROWS)],
                    out_hbm.at[pl.ds(base, ZROWS), pl.ds(c * DH, DH)])

  return scatter_kernel


def _dot(a, b):
  return lax.dot_general(a, b, (((1,), (0,)), ((), ())),
                         precision=lax.Precision.HIGHEST,
                         preferred_element_type=_f32)


def _mm_scale_body(x_ref, w_ref, degp_ref, xs_ref, dinv_ref):
  d = degp_ref[0, :, 0:1] + degp_ref[1, :, 0:1] + 1.0
  dinv = lax.rsqrt(d)
  xs_ref[...] = _dot(x_ref[...], w_ref[...]) * dinv
  dinv_ref[...] = dinv


def _tc_mm_scale(x, w, degp):
  return pl.pallas_call(
      _mm_scale_body,
      grid=(N // BN,),
      in_specs=[pl.BlockSpec((BN, D), lambda i: (i, 0)),
                pl.BlockSpec((D, D), lambda i: (0, 0)),
                pl.BlockSpec((NSC, BN, 16), lambda i: (0, i, 0))],
      out_specs=[pl.BlockSpec((BN, D), lambda i: (i, 0)),
                 pl.BlockSpec((BN, 1), lambda i: (i, 0))],
      out_shape=[jax.ShapeDtypeStruct((N, D), _f32),
                 jax.ShapeDtypeStruct((N, 1), _f32)],
  )(x, w, degp)


def _mid_body(y_ref, xs_ref, dinv_ref, b_ref, w_ref, o_ref):
  dinv = dinv_ref[...]
  h = (y_ref[...] + xs_ref[...]) * dinv + b_ref[...]
  o_ref[...] = _dot(h, w_ref[...]) * dinv


def _tc_mid(y, xs, dinv, b, w):
  return pl.pallas_call(
      _mid_body,
      grid=(N // BN,),
      in_specs=[pl.BlockSpec((BN, D), lambda i: (i, 0)),
                pl.BlockSpec((BN, D), lambda i: (i, 0)),
                pl.BlockSpec((BN, 1), lambda i: (i, 0)),
                pl.BlockSpec((1, D), lambda i: (0, 0)),
                pl.BlockSpec((D, D), lambda i: (0, 0))],
      out_specs=pl.BlockSpec((BN, D), lambda i: (i, 0)),
      out_shape=jax.ShapeDtypeStruct((N, D), _f32),
  )(y, xs, dinv, b, w)


def _fin_body(y_ref, xs_ref, dinv_ref, b_ref, o_ref):
  o_ref[...] = (y_ref[...] + xs_ref[...]) * dinv_ref[...] + b_ref[...]


def _tc_fin(y, xs, dinv, b):
  return pl.pallas_call(
      _fin_body,
      grid=(N // BN,),
      in_specs=[pl.BlockSpec((BN, D), lambda i: (i, 0)),
                pl.BlockSpec((BN, D), lambda i: (i, 0)),
                pl.BlockSpec((BN, 1), lambda i: (i, 0)),
                pl.BlockSpec((1, D), lambda i: (0, 0))],
      out_specs=pl.BlockSpec((BN, D), lambda i: (i, 0)),
      out_shape=jax.ShapeDtypeStruct((N, D), _f32),
  )(y, xs, dinv, b)


def kernel(in_feat, adj_t, W1, b1, W2, b2):
  e = adj_t.shape[1]
  if e % (NSUB * CH) or e % (NW * CH):
    raise ValueError("unsupported edge count for this kernel")
  nch = e // (NSUB * CH)
  nch32 = e // (NW * CH)

  adj4 = adj_t.reshape(2, NSUB, nch, CH)

  deg_k = _make_deg_kernel(nch32)
  scat_k = _make_scatter_kernel(nch)

  degp = deg_k(adj4)
  xs1, dinv = _tc_mm_scale(in_feat, W1, degp)
  y1 = scat_k(xs1.reshape(NSC * N, DH), adj4)
  xs2 = _tc_mid(y1, xs1, dinv, b1.reshape(1, D), W2)
  y2 = scat_k(xs2.reshape(NSC * N, DH), adj4)
  return _tc_fin(y2, xs2, dinv, b2.reshape(1, D))


# speedup vs baseline: 1.3824x; 1.0007x over previous
import functools

import jax
import jax.numpy as jnp
from jax import lax
from jax.experimental import pallas as pl
from jax.experimental.pallas import tpu as pltpu
from jax.experimental.pallas import tpu_sc as plsc

N = 10000
D = 128
DH = D // 2
NSC = 2
NSUB = 16
NW = NSC * NSUB
CH = 80
NACC = 10240
ZROWS = NACC // NSUB
BN = 2000

_mesh = plsc.VectorSubcoreMesh(
    core_axis_name="c", subcore_axis_name="s", num_cores=NSC)

_f32 = jnp.float32

_sc_params = pltpu.CompilerParams(use_tc_tiling_on_sc=False)


def _make_deg_kernel(nch):
  @functools.partial(
      pl.kernel,
      out_type=jax.ShapeDtypeStruct((NSC, NACC, 16), _f32),
      mesh=_mesh,
      scratch_types=[
          pltpu.VMEM((nch, CH), jnp.int32),
          pltpu.VMEM((CH, 16), _f32),
          pltpu.VMEM((128, 16), _f32),
          pltpu.SemaphoreType.DMA,
          pltpu.VMEM_SHARED((NACC, 16), _f32),
      ],
      compiler_params=_sc_params,
  )
  def deg_kernel(adj_hbm, out_hbm, dst_v, ones_v, zeros_v, sem, deg_sh):
    c = lax.axis_index("c")
    s = lax.axis_index("s")
    one = jnp.ones((16,), _f32)
    zero = jnp.zeros((16,), _f32)

    @pl.loop(0, CH)
    def _(r):
      ones_v[r, pl.ds(0, 16)] = one

    @pl.loop(0, 128)
    def _(r):
      zeros_v[r, pl.ds(0, 16)] = zero

    base = s * ZROWS
    for j in range(ZROWS // 128):
      pltpu.sync_copy(zeros_v, deg_sh.at[pl.ds(base + j * 128, 128)])
    pltpu.sync_copy(adj_hbm.at[1, s, pl.ds(c * nch, nch)], dst_v)
    plsc.subcore_barrier()

    NB = 8
    nfull = (nch // NB) * NB

    @pl.loop(0, nfull, step=NB)
    def _(cb):
      for j in range(NB):
        pltpu.async_copy(ones_v, deg_sh.at[dst_v.at[cb + j]], sem, add=True)
      for j in range(NB):
        pltpu.make_async_copy(ones_v, deg_sh.at[dst_v.at[0]], sem).wait()

    for ch in range(nfull, nch):
      pltpu.sync_copy(ones_v, deg_sh.at[dst_v.at[ch]], add=True)

    plsc.subcore_barrier()
    pltpu.sync_copy(deg_sh.at[pl.ds(base, ZROWS)],
                    out_hbm.at[c, pl.ds(base, ZROWS)])

  return deg_kernel


def _make_scatter_kernel(nch):
  G = 5
  PH = 50
  phases = [min(PH, nch - p * PH) for p in range(-(-nch // PH))]

  @functools.partial(
      pl.kernel,
      out_type=jax.ShapeDtypeStruct((NACC, D), _f32),
      mesh=_mesh,
      scratch_types=[
          pltpu.VMEM((PH, CH), jnp.int32),
          pltpu.VMEM((PH, CH), jnp.int32),
      ] + [pltpu.VMEM((CH, DH), _f32) for _ in range(2 * G)] + [
          pltpu.VMEM_SHARED((NACC, DH), _f32),
          pltpu.SemaphoreType.DMA,
          pltpu.SemaphoreType.DMA,
          pltpu.SemaphoreType.DMA,
          pltpu.SemaphoreType.DMA,
      ],
      compiler_params=_sc_params,
  )
  def scatter_kernel(xs_hbm, adj_hbm, out_hbm, src_v, dst_v, *rest):
    bufs = rest[:2 * G]
    seta, setb = bufs[:G], bufs[G:]
    acc_sh, gsa, gsb, ssa, ssb = rest[2 * G:]
    c = lax.axis_index("c")
    s = lax.axis_index("s")
    cvec = lax.broadcast(c, (16,))
    zero = jnp.zeros((16,), _f32)
    buf0 = seta[0]

    @pl.loop(0, CH)
    def _(r):
      for k in range(DH // 16):
        buf0[r, pl.ds(k * 16, 16)] = zero

    base = s * ZROWS
    for j in range(ZROWS // CH):
      pltpu.sync_copy(buf0, acc_sh.at[pl.ds(base + j * CH, CH)])
    plsc.subcore_barrier()

    def fire_g(cb, st, sem):
      for j in range(G):
        pltpu.async_copy(xs_hbm.at[src_v.at[cb + j]], st[j], sem)

    def wait_g(st, sem):
      for j in range(G):
        pltpu.make_async_copy(xs_hbm.at[src_v.at[0]], st[j], sem).wait()

    def fire_s(cb, st, sem):
      for j in range(G):
        pltpu.async_copy(st[j], acc_sh.at[dst_v.at[cb + j]], sem, add=True)

    def wait_s(st, sem):
      for j in range(G):
        pltpu.make_async_copy(st[j], acc_sh.at[dst_v.at[0]], sem).wait()

    for p, plen in enumerate(phases):
      pltpu.sync_copy(adj_hbm.at[0, s, pl.ds(p * PH, plen)],
                      src_v.at[pl.ds(0, plen)])
      pltpu.sync_copy(adj_hbm.at[1, s, pl.ds(p * PH, plen)],
                      dst_v.at[pl.ds(0, plen)])

      @pl.loop(0, plen)
      def _(rr):
        for k in range(CH // 16):
          v = src_v[rr, pl.ds(k * 16, 16)]
          src_v[rr, pl.ds(k * 16, 16)] = v + v + cvec

      ngrp = (plen // (2 * G)) * 2
      if ngrp >= 4:
        fire_g(0, seta, gsa)
        fire_g(G, setb, gsb)
        wait_g(seta, gsa)
        fire_s(0, seta, ssa)
        wait_g(setb, gsb)
        fire_s(G, setb, ssb)
        wait_s(seta, ssa)
        fire_g(2 * G, seta, gsa)

        @pl.loop(2 * G, ngrp * G, step=2 * G)
        def _(cb):
          wait_g(seta, gsa)
          fire_s(cb, seta, ssa)
          wait_s(setb, ssb)
          fire_g(cb + G, setb, gsb)
          wait_g(setb, gsb)
          fire_s(cb + G, setb, ssb)
          wait_s(seta, ssa)

          @pl.when(cb + 2 * G < ngrp * G)
          def _():
            fire_g(cb + 2 * G, seta, gsa)

        wait_s(setb, ssb)
        done = ngrp * G
      else:
        done = 0

      for ch0 in range(done, plen):
        st = seta[ch0 % G]
        pltpu.async_copy(xs_hbm.at[src_v.at[ch0]], st, gsa).wait()
        pltpu.sync_copy(st, acc_sh.at[dst_v.at[ch0]], add=True)

    plsc.subcore_barrier()
    pltpu.sync_copy(acc_sh.at[pl.ds(base, ZROWS)],
                    out_hbm.at[pl.ds(base, ZROWS), pl.ds(c * DH, DH)])

  return scatter_kernel


def _dot(a, b):
  return lax.dot_general(a, b, (((1,), (0,)), ((), ())),
                         precision=lax.Precision.HIGHEST,
                         preferred_element_type=_f32)


def _mm_scale_body(x_ref, w_ref, degp_ref, xs_ref, dinv_ref):
  d = degp_ref[0, :, 0:1] + degp_ref[1, :, 0:1] + 1.0
  dinv = lax.rsqrt(d)
  xs_ref[...] = _dot(x_ref[...], w_ref[...]) * dinv
  dinv_ref[...] = dinv


def _tc_mm_scale(x, w, degp):
  return pl.pallas_call(
      _mm_scale_body,
      grid=(N // BN,),
      in_specs=[pl.BlockSpec((BN, D), lambda i: (i, 0)),
                pl.BlockSpec((D, D), lambda i: (0, 0)),
                pl.BlockSpec((NSC, BN, 16), lambda i: (0, i, 0))],
      out_specs=[pl.BlockSpec((BN, D), lambda i: (i, 0)),
                 pl.BlockSpec((BN, 1), lambda i: (i, 0))],
      out_shape=[jax.ShapeDtypeStruct((N, D), _f32),
                 jax.ShapeDtypeStruct((N, 1), _f32)],
  )(x, w, degp)


def _mid_body(y_ref, xs_ref, dinv_ref, b_ref, w_ref, o_ref):
  dinv = dinv_ref[...]
  h = (y_ref[...] + xs_ref[...]) * dinv + b_ref[...]
  o_ref[...] = _dot(h, w_ref[...]) * dinv


def _tc_mid(y, xs, dinv, b, w):
  return pl.pallas_call(
      _mid_body,
      grid=(N // BN,),
      in_specs=[pl.BlockSpec((BN, D), lambda i: (i, 0)),
                pl.BlockSpec((BN, D), lambda i: (i, 0)),
                pl.BlockSpec((BN, 1), lambda i: (i, 0)),
                pl.BlockSpec((1, D), lambda i: (0, 0)),
                pl.BlockSpec((D, D), lambda i: (0, 0))],
      out_specs=pl.BlockSpec((BN, D), lambda i: (i, 0)),
      out_shape=jax.ShapeDtypeStruct((N, D), _f32),
  )(y, xs, dinv, b, w)


def _fin_body(y_ref, xs_ref, dinv_ref, b_ref, o_ref):
  o_ref[...] = (y_ref[...] + xs_ref[...]) * dinv_ref[...] + b_ref[...]


def _tc_fin(y, xs, dinv, b):
  return pl.pallas_call(
      _fin_body,
      grid=(N // BN,),
      in_specs=[pl.BlockSpec((BN, D), lambda i: (i, 0)),
                pl.BlockSpec((BN, D), lambda i: (i, 0)),
                pl.BlockSpec((BN, 1), lambda i: (i, 0)),
                pl.BlockSpec((1, D), lambda i: (0, 0))],
      out_specs=pl.BlockSpec((BN, D), lambda i: (i, 0)),
      out_shape=jax.ShapeDtypeStruct((N, D), _f32),
  )(y, xs, dinv, b)


def kernel(in_feat, adj_t, W1, b1, W2, b2):
  e = adj_t.shape[1]
  if e % (NSUB * CH) or e % (NW * CH):
    raise ValueError("unsupported edge count for this kernel")
  nch = e // (NSUB * CH)
  nch32 = e // (NW * CH)

  adj4 = adj_t.reshape(2, NSUB, nch, CH)

  deg_k = _make_deg_kernel(nch32)
  scat_k = _make_scatter_kernel(nch)

  degp = deg_k(adj4)
  xs1, dinv = _tc_mm_scale(in_feat, W1, degp)
  y1 = scat_k(xs1.reshape(NSC * N, DH), adj4)
  xs2 = _tc_mid(y1, xs1, dinv, b1.reshape(1, D), W2)
  y2 = scat_k(xs2.reshape(NSC * N, DH), adj4)
  return _tc_fin(y2, xs2, dinv, b2.reshape(1, D))
